# Initial kernel scaffold; baseline (speedup 1.0000x reference)
import jax
import jax.numpy as jnp
from jax.experimental import pallas as pl


def kernel(atom_state, bond_state, connectivity, W1, b1, W2, b2):
    raise NotImplementedError("write your pallas kernel here")


# speedup vs baseline: 7102.7915x; 7102.7915x over previous
import functools

import jax
import jax.numpy as jnp
from jax import lax
from jax.experimental import pallas as pl
from jax.experimental.pallas import tpu as pltpu
from jax.experimental.pallas import tpu_sc as plsc

N_NODES = 10000
N_EDGES = 160000
D = 256

_INFO = plsc.get_sparse_core_info()
_NW = _INFO.num_cores * _INFO.num_subcores
_EPW = N_EDGES // _NW
_C = 40
_NCHUNK = _EPW // _C


def _sc_gather(table_hbm, idx_src_hbm, idx_tgt_hbm, src_out, tgt_out,
               idx_src_v, idx_tgt_v, rows_s, rows_t, sem_s, sem_t):
    wid = lax.axis_index("s") * _INFO.num_cores + lax.axis_index("c")
    base = wid * _EPW
    pltpu.sync_copy(idx_src_hbm.at[pl.ds(base, _EPW)], idx_src_v)
    pltpu.sync_copy(idx_tgt_hbm.at[pl.ds(base, _EPW)], idx_tgt_v)

    def body(c, carry):
        off = c * _C
        g_s = pltpu.async_copy(table_hbm.at[idx_src_v.at[pl.ds(off, _C)]],
                               rows_s, sem_s)
        g_t = pltpu.async_copy(table_hbm.at[idx_tgt_v.at[pl.ds(off, _C)]],
                               rows_t, sem_t)
        g_s.wait()
        pltpu.sync_copy(rows_s, src_out.at[pl.ds(base + off, _C)])
        g_t.wait()
        pltpu.sync_copy(rows_t, tgt_out.at[pl.ds(base + off, _C)])
        return carry

    lax.fori_loop(0, _NCHUNK, body, 0)


@functools.partial(jax.jit, static_argnums=())
def _gather_rows(table, idx_src, idx_tgt):
    mesh = plsc.VectorSubcoreMesh(core_axis_name="c", subcore_axis_name="s")
    f = functools.partial(
        pl.kernel,
        out_type=[jax.ShapeDtypeStruct((N_EDGES, D), jnp.float32),
                  jax.ShapeDtypeStruct((N_EDGES, D), jnp.float32)],
        mesh=mesh,
        scratch_types=[
            pltpu.VMEM((_EPW,), jnp.int32),
            pltpu.VMEM((_EPW,), jnp.int32),
            pltpu.VMEM((_C, D), jnp.float32),
            pltpu.VMEM((_C, D), jnp.float32),
            pltpu.SemaphoreType.DMA,
            pltpu.SemaphoreType.DMA,
        ],
    )(_sc_gather)
    return f(table, idx_src, idx_tgt)


_M = 2000


def _mm_body(bond_ref, src_ref, tgt_ref, w1a_ref, w1b_ref, w1c_ref,
             b1_ref, w2_ref, b2_ref, out_ref):
    acc = jnp.dot(bond_ref[...], w1a_ref[...],
                  preferred_element_type=jnp.float32)
    acc = acc + jnp.dot(src_ref[...], w1b_ref[...],
                        preferred_element_type=jnp.float32)
    acc = acc + jnp.dot(tgt_ref[...], w1c_ref[...],
                        preferred_element_type=jnp.float32)
    h = jnp.maximum(acc + b1_ref[...], 0.0)
    out_ref[...] = jnp.dot(h, w2_ref[...],
                           preferred_element_type=jnp.float32) + b2_ref[...]


def _dense(bond, src, tgt, w1a, w1b, w1c, b1, w2, b2):
    grid = (N_EDGES // _M,)
    row_spec = pl.BlockSpec((_M, D), lambda i: (i, 0))
    full = lambda shape: pl.BlockSpec(shape, lambda i: (0, 0))
    return pl.pallas_call(
        _mm_body,
        grid=grid,
        in_specs=[row_spec, row_spec, row_spec,
                  full((D, 2 * D)), full((D, 2 * D)), full((D, 2 * D)),
                  full((1, 2 * D)), full((2 * D, D)), full((1, D))],
        out_specs=row_spec,
        out_shape=jax.ShapeDtypeStruct((N_EDGES, D), jnp.float32),
    )(bond, src, tgt, w1a, w1b, w1c, b1, w2, b2)


def kernel(atom_state, bond_state, connectivity, W1, b1, W2, b2):
    table = atom_state[0]
    bond = bond_state[0]
    idx_tgt = connectivity[0, :, 0]
    idx_src = connectivity[0, :, 1]
    src, tgt = _gather_rows(table, idx_src, idx_tgt)
    w1a, w1b, w1c = W1[:D], W1[D:2 * D], W1[2 * D:]
    out = _dense(bond, src, tgt, w1a, w1b, w1c,
                 b1.reshape(1, 2 * D), W2, b2.reshape(1, D))
    return out[None]
